# baseline (speedup 1.0000x reference)
import functools

import jax
import jax.numpy as jnp
from jax import lax
from jax.experimental import pallas as pl
from jax.experimental.pallas import tpu as pltpu
from jax.experimental.pallas import tpu_sc as plsc

N = 16384
K = 1024
D = 64
BETA = 0.25
BLK = 2048
NB = N // BLK

_NC = 2
_NS = 16
_NW = _NC * _NS
_ROWS_PER_W = N // _NW
_CHUNK = 128
_NCHUNK = _ROWS_PER_W // _CHUNK


def _vq_body(xt_ref, pt_ref, x2_ref, p2_ref, iota_ref, idx_ref, vq_ref,
             ent_ref, acc_ref, vqacc_ref):
    i = pl.program_id(0)
    xt = xt_ref[...]
    pt = pt_ref[...]
    dots = lax.dot_general(
        xt, pt, (((0,), (0,)), ((), ())),
        preferred_element_type=jnp.float32)
    x2 = x2_ref[...].reshape(BLK, 1)
    dists = (x2 + p2_ref[...]) + dots
    mind = jnp.min(dists, axis=1, keepdims=True)
    idx_f = jnp.min(jnp.where(dists == mind, iota_ref[...], float(K)),
                    axis=1, keepdims=True)
    idx_ref[...] = idx_f.astype(jnp.int32).reshape(BLK // 128, 128)

    e = jnp.exp(mind - dists)
    z = jnp.sum(e, axis=1, keepdims=True)

    @pl.when(i == 0)
    def _init():
        acc_ref[...] = jnp.zeros_like(acc_ref)
        vqacc_ref[0, 0] = 0.0

    acc_ref[...] += jnp.sum(e * (1.0 / z), axis=0, keepdims=True)
    vqacc_ref[0, 0] += jnp.sum(mind)

    @pl.when(i == NB - 1)
    def _fin():
        s = acc_ref[...] * (1.0 / N) + 1e-8
        s = s / jnp.sum(s)
        ent_ref[...] = jnp.sum(-s * jnp.log(s), keepdims=True).reshape(1, 1)
        vq_ref[...] = jnp.full(
            (1, 1), (1.0 + BETA) * vqacc_ref[0, 0] / (N * D), jnp.float32)


_vq_call = pl.pallas_call(
    _vq_body,
    grid=(NB,),
    in_specs=[
        pl.BlockSpec((D, BLK), lambda i: (0, i)),
        pl.BlockSpec((D, K), lambda i: (0, 0)),
        pl.BlockSpec((BLK,), lambda i: (i,)),
        pl.BlockSpec((1, K), lambda i: (0, 0)),
        pl.BlockSpec((1, K), lambda i: (0, 0)),
    ],
    out_specs=[
        pl.BlockSpec((BLK // 128, 128), lambda i: (i, 0)),
        pl.BlockSpec((1, 1), lambda i: (0, 0)),
        pl.BlockSpec((1, 1), lambda i: (0, 0)),
    ],
    out_shape=[
        jax.ShapeDtypeStruct((N // 128, 128), jnp.int32),
        jax.ShapeDtypeStruct((1, 1), jnp.float32),
        jax.ShapeDtypeStruct((1, 1), jnp.float32),
    ],
    scratch_shapes=[
        pltpu.VMEM((1, K), jnp.float32),
        pltpu.SMEM((1, 1), jnp.float32),
    ],
)


@functools.cache
def _sc_gather_call():
    @functools.partial(
        pl.kernel,
        mesh=plsc.VectorSubcoreMesh(core_axis_name="c", subcore_axis_name="s"),
        out_type=jax.ShapeDtypeStruct((D, N), jnp.float32),
        scratch_types=[
            pltpu.VMEM((_NCHUNK, _CHUNK), jnp.int32),
            pltpu.VMEM((_ROWS_PER_W,), jnp.int32),
            pltpu.VMEM((K * D,), jnp.float32),
            pltpu.VMEM((D, _ROWS_PER_W), jnp.float32),
        ],
        compiler_params=pltpu.CompilerParams(needs_layout_passes=False),
    )
    def _sc_gather(table_hbm, idx_hbm, out_hbm, idx2_v, idx_v, tab_v, out_v):
        wid = lax.axis_index("s") * _NC + lax.axis_index("c")
        pltpu.sync_copy(idx_hbm.at[pl.ds(wid * _NCHUNK, _NCHUNK)], idx2_v)
        pltpu.sync_copy(table_hbm, tab_v)
        for r in range(_NCHUNK):
            for c in range(_CHUNK // 16):
                idx_v[pl.ds((r * (_CHUNK // 16) + c) * 16, 16)] = (
                    idx2_v[r, pl.ds(c * 16, 16)])

        def body(k, carry):
            base = idx_v[pl.ds(k * 16, 16)] * D
            for d in range(D):
                out_v[d, pl.ds(k * 16, 16)] = plsc.load_gather(
                    tab_v, [base + d])
            return carry

        lax.fori_loop(0, _ROWS_PER_W // 16, body, 0)
        pltpu.sync_copy(
            out_v, out_hbm.at[:, pl.ds(wid * _ROWS_PER_W, _ROWS_PER_W)])

    return _sc_gather


def kernel(latents, prototypes):
    x2 = jnp.sum(latents ** 2, axis=1)
    p2 = jnp.sum(prototypes ** 2, axis=1).reshape(1, K)
    iota_row = lax.broadcasted_iota(jnp.float32, (1, K), 1)
    idx, vq, ent = _vq_call(latents.T, -2.0 * prototypes.T, x2, p2, iota_row)
    gathered_t = _sc_gather_call()(prototypes.reshape(K * D), idx)
    return gathered_t.T, vq[0, 0], ent[0, 0]


# speedup vs baseline: 1.1409x; 1.1409x over previous
import functools

import jax
import jax.numpy as jnp
from jax import lax
from jax.experimental import pallas as pl
from jax.experimental.pallas import tpu as pltpu
from jax.experimental.pallas import tpu_sc as plsc

N = 16384
K = 1024
D = 64
BETA = 0.25
BLK = 2048
NB = N // BLK

_NC = 2
_NS = 16
_NW = _NC * _NS
_ROWS_PER_W = N // _NW
_CHUNK = 128
_NCHUNK = _ROWS_PER_W // _CHUNK


def _vq_body(xt_ref, pt_ref, x2_ref, p2_ref, iota_ref, idx_ref, vq_ref,
             ent_ref, acc_ref, vqacc_ref):
    i = pl.program_id(0)
    xt = xt_ref[...]
    pt = pt_ref[...]
    dots = lax.dot_general(
        xt, pt, (((0,), (0,)), ((), ())),
        preferred_element_type=jnp.float32)
    x2 = x2_ref[...].reshape(BLK, 1)
    dists = (x2 + p2_ref[...]) + dots
    mind = jnp.min(dists, axis=1, keepdims=True)
    idx_f = jnp.min(jnp.where(dists == mind, iota_ref[...], float(K)),
                    axis=1, keepdims=True)
    idx_ref[...] = idx_f.astype(jnp.int32).reshape(BLK // 128, 128)

    e = jnp.exp(mind - dists)
    z = jnp.sum(e, axis=1, keepdims=True)

    @pl.when(i == 0)
    def _init():
        acc_ref[...] = jnp.zeros_like(acc_ref)
        vqacc_ref[0, 0] = 0.0

    acc_ref[...] += jnp.sum(e * (1.0 / z), axis=0, keepdims=True)
    vqacc_ref[0, 0] += jnp.sum(mind)

    @pl.when(i == NB - 1)
    def _fin():
        s = acc_ref[...] * (1.0 / N) + 1e-8
        s = s / jnp.sum(s)
        ent_ref[...] = jnp.sum(-s * jnp.log(s), keepdims=True).reshape(1, 1)
        vq_ref[...] = jnp.full(
            (1, 1), (1.0 + BETA) * vqacc_ref[0, 0] / (N * D), jnp.float32)


_vq_call = pl.pallas_call(
    _vq_body,
    grid=(NB,),
    in_specs=[
        pl.BlockSpec((D, BLK), lambda i: (0, i)),
        pl.BlockSpec((D, K), lambda i: (0, 0)),
        pl.BlockSpec((BLK,), lambda i: (i,)),
        pl.BlockSpec((1, K), lambda i: (0, 0)),
        pl.BlockSpec((1, K), lambda i: (0, 0)),
    ],
    out_specs=[
        pl.BlockSpec((BLK // 128, 128), lambda i: (i, 0)),
        pl.BlockSpec((1, 1), lambda i: (0, 0)),
        pl.BlockSpec((1, 1), lambda i: (0, 0)),
    ],
    out_shape=[
        jax.ShapeDtypeStruct((N // 128, 128), jnp.int32),
        jax.ShapeDtypeStruct((1, 1), jnp.float32),
        jax.ShapeDtypeStruct((1, 1), jnp.float32),
    ],
    scratch_shapes=[
        pltpu.VMEM((1, K), jnp.float32),
        pltpu.SMEM((1, 1), jnp.float32),
    ],
)


@functools.cache
def _sc_gather_call():
    @functools.partial(
        pl.kernel,
        mesh=plsc.VectorSubcoreMesh(core_axis_name="c", subcore_axis_name="s"),
        out_type=jax.ShapeDtypeStruct((N, 2 * D), jnp.float32),
        scratch_types=[
            pltpu.VMEM((_NCHUNK, _CHUNK), jnp.int32),
            pltpu.VMEM((_ROWS_PER_W, 2 * D), jnp.float32),
            pltpu.SemaphoreType.DMA,
        ],
    )
    def _sc_gather(table_hbm, idx_hbm, out_hbm, idx_v, rows_v, sem):
        wid = lax.axis_index("s") * _NC + lax.axis_index("c")
        pltpu.sync_copy(idx_hbm.at[pl.ds(wid * _NCHUNK, _NCHUNK)], idx_v)
        copies = []
        for j in range(_NCHUNK):
            copies.append(pltpu.async_copy(
                table_hbm.at[idx_v.at[j]],
                rows_v.at[pl.ds(j * _CHUNK, _CHUNK)], sem))
        for c in copies:
            c.wait()
        pltpu.sync_copy(
            rows_v, out_hbm.at[pl.ds(wid * _ROWS_PER_W, _ROWS_PER_W)])

    return _sc_gather


def kernel(latents, prototypes):
    x2 = jnp.sum(latents ** 2, axis=1)
    p2 = jnp.sum(prototypes ** 2, axis=1).reshape(1, K)
    iota_row = lax.broadcasted_iota(jnp.float32, (1, K), 1)
    idx, vq, ent = _vq_call(latents.T, -2.0 * prototypes.T, x2, p2, iota_row)
    table = jnp.pad(prototypes, ((0, 0), (0, D)))
    gathered = _sc_gather_call()(table, idx)
    return gathered[:, :D], vq[0, 0], ent[0, 0]
